# baseline (speedup 1.0000x reference)
import functools

import jax
import jax.numpy as jnp
from jax import lax
from jax.experimental import pallas as pl
from jax.experimental.pallas import tpu as pltpu

Q = 16
DIM = 64
PACK = 4
KTOP = 10
NSEL = 24
NCAND = 64
BUF = 128
EPS = 1.5
IDXB = 15
IDXM = (1 << IDXB) - 1


def _fold_slots(x):
    out = x[:, 0:Q]
    for s in range(1, PACK):
        out = jnp.minimum(out, x[:, s * Q:(s + 1) * Q])
    return out


def _scan_block(kb, lidx, a1, a2, base, dout_ref, iout_ref, iscr_ref):
    kbb = kb.astype(jnp.bfloat16)
    ksqb = (kb * kb).astype(jnp.bfloat16)
    qk = lax.dot_general(kbb, a1, (((1,), (0,)), ((), ())),
                         preferred_element_type=jnp.float32)
    c2 = lax.dot_general(ksqb, a2, (((1,), (0,)), ((), ())),
                         preferred_element_type=jnp.float32)
    d = c2 + qk

    s32 = lax.bitcast_convert_type(d, jnp.int32)
    key = s32 ^ (lax.shift_right_arithmetic(s32, 31) &
                 jnp.int32(0x7FFFFFFF))
    comb = (key & jnp.int32(~IDXM)) | lidx
    iscr_ref[...] = comb

    bufi = lax.broadcasted_iota(jnp.int32, (BUF, Q), 0)
    MAXI = jnp.int32(2**31 - 1)

    def cond(c):
        return (c[0] < NSEL) & c[1]

    def body(c):
        r, _, prev = c
        cc = iscr_ref[...]
        prev4 = jnp.concatenate([prev] * PACK, axis=1)
        live = jnp.where(cc > prev4, cc, MAXI)
        mcol = jnp.min(live, axis=0, keepdims=True)
        mc = _fold_slots(mcol)
        kbits = mc & jnp.int32(~IDXM)
        dec = lax.bitcast_convert_type(
            kbits ^ (lax.shift_right_arithmetic(kbits, 31) &
                     jnp.int32(0x7FFFFFFF)), jnp.float32)
        gq = (mc & IDXM) + base

        vals = dout_ref[...]
        idxs = iout_ref[...]
        do_q = dec < vals[KTOP - 1:KTOP, :] + EPS
        pos = jnp.sum((vals <= dec).astype(jnp.int32),
                      axis=0, keepdims=True)
        vshift = jnp.concatenate([vals[:1], vals[:-1]], axis=0)
        ishift = jnp.concatenate([idxs[:1], idxs[:-1]], axis=0)
        newv = jnp.where(bufi < pos, vals,
                         jnp.where(bufi == pos, dec, vshift))
        newi = jnp.where(bufi < pos, idxs,
                         jnp.where(bufi == pos, gq, ishift))
        dout_ref[...] = jnp.where(do_q, newv, vals)
        iout_ref[...] = jnp.where(do_q, newi, idxs)
        return r + jnp.int32(1), jnp.any(do_q), mc

    lax.while_loop(cond, body,
                   (jnp.int32(0), True,
                    jnp.full((1, Q), -(2**31 - 1) - 1, jnp.int32)))


def _knn_kernel64(a1_ref, a2_ref, k_ref, dout_ref, iout_ref, iscr_ref,
                  *, block_k, key_offset):
    t = pl.program_id(0)
    rows = block_k // PACK

    @pl.when(t == 0)
    def _init():
        dout_ref[...] = jnp.full((BUF, Q), jnp.inf, jnp.float32)
        iout_ref[...] = jnp.zeros((BUF, Q), jnp.int32)

    kb = jnp.concatenate(
        [k_ref[s * rows:(s + 1) * rows, :] for s in range(PACK)],
        axis=1)
    rowi = lax.broadcasted_iota(jnp.int32, (rows, PACK * Q), 0)
    slot = lax.broadcasted_iota(jnp.int32, (rows, PACK * Q), 1) // Q
    lidx = slot * rows + rowi
    base = (t * block_k).astype(jnp.int32) + key_offset
    _scan_block(kb, lidx, a1_ref[...], a2_ref[...], base,
                dout_ref, iout_ref, iscr_ref)


def _knn_kernel128(a1_ref, a2_ref, k_ref, dout_ref, iout_ref, iscr_ref,
                   *, block_r, key_offset):
    t = pl.program_id(0)
    half = block_r // 2

    @pl.when(t == 0)
    def _init():
        dout_ref[...] = jnp.full((BUF, Q), jnp.inf, jnp.float32)
        iout_ref[...] = jnp.zeros((BUF, Q), jnp.int32)

    kb = jnp.concatenate([k_ref[0:half, :], k_ref[half:block_r, :]],
                         axis=1)
    rowi = lax.broadcasted_iota(jnp.int32, (half, PACK * Q), 0)
    slot = lax.broadcasted_iota(jnp.int32, (half, PACK * Q), 1) // Q
    lidx = 2 * rowi + (slot % 2) + (slot // 2) * block_r
    base = (t * (2 * block_r)).astype(jnp.int32) + key_offset
    _scan_block(kb, lidx, a1_ref[...], a2_ref[...], base,
                dout_ref, iout_ref, iscr_ref)


def kernel(queries, keys, k):
    nkeys = keys.shape[0]
    half_keys = nkeys // 2

    eye = jnp.eye(PACK, dtype=jnp.float32)
    a1 = jnp.einsum("st,dq->sdtq", eye, -2.0 * queries.T).reshape(
        PACK * DIM, PACK * Q).astype(jnp.bfloat16)
    a2 = jnp.einsum("st,dq->sdtq", eye,
                    jnp.ones((DIM, Q), jnp.float32)).reshape(
        PACK * DIM, PACK * Q).astype(jnp.bfloat16)

    out_common = dict(
        out_specs=[
            pl.BlockSpec((BUF, Q), lambda t: (0, 0)),
            pl.BlockSpec((BUF, Q), lambda t: (0, 0)),
        ],
        out_shape=[
            jax.ShapeDtypeStruct((BUF, Q), jnp.float32),
            jax.ShapeDtypeStruct((BUF, Q), jnp.int32),
        ],
    )

    keys_b = keys[half_keys:].reshape(half_keys // 2, 2 * DIM)

    block_k = 20000
    assert half_keys % block_k == 0 and block_k <= (1 << IDXB)
    nb_a = half_keys // block_k
    _, ipad_a = pl.pallas_call(
        functools.partial(_knn_kernel64, block_k=block_k, key_offset=0),
        grid=(nb_a,),
        in_specs=[
            pl.BlockSpec((PACK * DIM, PACK * Q), lambda t: (0, 0)),
            pl.BlockSpec((PACK * DIM, PACK * Q), lambda t: (0, 0)),
            pl.BlockSpec((block_k, DIM), lambda t: (t, 0)),
        ],
        scratch_shapes=[pltpu.VMEM((block_k // PACK, PACK * Q), jnp.int32)],
        **out_common,
    )(a1, a2, keys[:half_keys])

    block_r = 10000
    nrows_b = half_keys // 2
    assert nrows_b % block_r == 0 and 2 * block_r <= (1 << IDXB)
    nb_b = nrows_b // block_r
    _, ipad_b = pl.pallas_call(
        functools.partial(_knn_kernel128, block_r=block_r,
                          key_offset=half_keys),
        grid=(nb_b,),
        in_specs=[
            pl.BlockSpec((PACK * DIM, PACK * Q), lambda t: (0, 0)),
            pl.BlockSpec((PACK * DIM, PACK * Q), lambda t: (0, 0)),
            pl.BlockSpec((block_r, 2 * DIM), lambda t: (t, 0)),
        ],
        scratch_shapes=[pltpu.VMEM((block_r // 2, PACK * Q), jnp.int32)],
        **out_common,
    )(a1, a2, keys_b)

    cand = jnp.sort(jnp.concatenate(
        [ipad_a[:NCAND, :].reshape(-1), ipad_b[:NCAND, :].reshape(-1)]))
    dup = jnp.concatenate(
        [jnp.zeros((1,), jnp.bool_), cand[1:] == cand[:-1]])
    sub = keys[cand]
    q2 = jnp.sum(queries * queries, axis=1, keepdims=True)
    c2 = jnp.sum(sub * sub, axis=1)[None, :]
    d2 = q2 + c2 - 2.0 * (queries @ sub.T)
    d2 = jnp.where(dup[None, :], jnp.inf, d2)
    neg_vals, pos = lax.top_k(-d2, KTOP)
    D = -neg_vals
    I = cand[pos]
    kth = D[-1, -1]
    return D, I, kth


# speedup vs baseline: 1.6114x; 1.6114x over previous
import functools

import jax
import jax.numpy as jnp
from jax import lax
from jax.experimental import pallas as pl
from jax.experimental.pallas import tpu as pltpu

Q = 16
DIM = 64
PACK = 4
KTOP = 10
NSEL = 24
NCAND = 64
BUF = 128
EPS = 1.5
IDXB = 15
IDXM = (1 << IDXB) - 1


def _fold_slots(x):
    out = x[:, 0:Q]
    for s in range(1, PACK):
        out = jnp.minimum(out, x[:, s * Q:(s + 1) * Q])
    return out


def _knn_kernel(a1_ref, a2_ref, k_ref, dout_ref, iout_ref, iscr_ref,
                *, block_k):
    t = pl.program_id(0)
    rows = block_k // PACK

    @pl.when(t == 0)
    def _init():
        dout_ref[...] = jnp.full((BUF, Q), jnp.inf, jnp.float32)
        iout_ref[...] = jnp.zeros((BUF, Q), jnp.int32)

    kb = jnp.concatenate(
        [k_ref[s * rows:(s + 1) * rows, :] for s in range(PACK)],
        axis=1)
    a1 = a1_ref[...]
    a2 = a2_ref[...]

    kbb = kb.astype(jnp.bfloat16)
    ksqb = (kb * kb).astype(jnp.bfloat16)
    qk = lax.dot_general(kbb, a1, (((1,), (0,)), ((), ())),
                         preferred_element_type=jnp.float32)
    c2 = lax.dot_general(ksqb, a2, (((1,), (0,)), ((), ())),
                         preferred_element_type=jnp.float32)
    d = c2 + qk

    rowi = lax.broadcasted_iota(jnp.int32, (rows, PACK * Q), 0)
    slot = lax.broadcasted_iota(jnp.int32, (rows, PACK * Q), 1) // Q
    lidx = slot * rows + rowi
    s32 = lax.bitcast_convert_type(d, jnp.int32)
    key = s32 ^ (lax.shift_right_arithmetic(s32, 31) &
                 jnp.int32(0x7FFFFFFF))
    comb = (key & jnp.int32(~IDXM)) | lidx
    iscr_ref[...] = comb

    base = (t * block_k).astype(jnp.int32)
    bufi = lax.broadcasted_iota(jnp.int32, (BUF, Q), 0)
    MAXI = jnp.int32(2**31 - 1)

    def cond(c):
        return (c[0] < NSEL) & c[1]

    def body(c):
        r, _, prev = c
        cc = iscr_ref[...]
        prev4 = jnp.concatenate([prev] * PACK, axis=1)
        live = jnp.where(cc > prev4, cc, MAXI)
        mcol = jnp.min(live, axis=0, keepdims=True)
        mc = _fold_slots(mcol)
        kbits = mc & jnp.int32(~IDXM)
        dec = lax.bitcast_convert_type(
            kbits ^ (lax.shift_right_arithmetic(kbits, 31) &
                     jnp.int32(0x7FFFFFFF)), jnp.float32)
        gq = (mc & IDXM) + base

        vals = dout_ref[...]
        idxs = iout_ref[...]
        do_q = dec < vals[KTOP - 1:KTOP, :] + EPS
        pos = jnp.sum((vals <= dec).astype(jnp.int32),
                      axis=0, keepdims=True)
        vshift = jnp.concatenate([vals[:1], vals[:-1]], axis=0)
        ishift = jnp.concatenate([idxs[:1], idxs[:-1]], axis=0)
        newv = jnp.where(bufi < pos, vals,
                         jnp.where(bufi == pos, dec, vshift))
        newi = jnp.where(bufi < pos, idxs,
                         jnp.where(bufi == pos, gq, ishift))
        dout_ref[...] = jnp.where(do_q, newv, vals)
        iout_ref[...] = jnp.where(do_q, newi, idxs)
        return r + jnp.int32(1), jnp.any(do_q), mc

    lax.while_loop(cond, body,
                   (jnp.int32(0), True,
                    jnp.full((1, Q), -(2**31 - 1) - 1, jnp.int32)))


def kernel(queries, keys, k):
    nkeys = keys.shape[0]
    block_k = 20000
    assert nkeys % block_k == 0
    assert block_k <= (1 << IDXB)
    nb = nkeys // block_k
    rows = block_k // PACK

    eye = jnp.eye(PACK, dtype=jnp.float32)
    a1 = jnp.einsum("st,dq->sdtq", eye, -2.0 * queries.T).reshape(
        PACK * DIM, PACK * Q).astype(jnp.bfloat16)
    a2 = jnp.einsum("st,dq->sdtq", eye,
                    jnp.ones((DIM, Q), jnp.float32)).reshape(
        PACK * DIM, PACK * Q).astype(jnp.bfloat16)

    _, ipad = pl.pallas_call(
        functools.partial(_knn_kernel, block_k=block_k),
        grid=(nb,),
        in_specs=[
            pl.BlockSpec((PACK * DIM, PACK * Q), lambda t: (0, 0)),
            pl.BlockSpec((PACK * DIM, PACK * Q), lambda t: (0, 0)),
            pl.BlockSpec((block_k, DIM), lambda t: (t, 0)),
        ],
        out_specs=[
            pl.BlockSpec((BUF, Q), lambda t: (0, 0)),
            pl.BlockSpec((BUF, Q), lambda t: (0, 0)),
        ],
        out_shape=[
            jax.ShapeDtypeStruct((BUF, Q), jnp.float32),
            jax.ShapeDtypeStruct((BUF, Q), jnp.int32),
        ],
        scratch_shapes=[pltpu.VMEM((rows, PACK * Q), jnp.int32)],
    )(a1, a2, keys)

    cand = jnp.sort(ipad[:NCAND, :].reshape(-1))
    dup = jnp.concatenate(
        [jnp.zeros((1,), jnp.bool_), cand[1:] == cand[:-1]])
    sub = keys[cand]
    q2 = jnp.sum(queries * queries, axis=1, keepdims=True)
    c2 = jnp.sum(sub * sub, axis=1)[None, :]
    d2 = q2 + c2 - 2.0 * (queries @ sub.T)
    d2 = jnp.where(dup[None, :], jnp.inf, d2)
    neg_vals, pos = lax.top_k(-d2, KTOP)
    D = -neg_vals
    I = cand[pos]
    kth = D[-1, -1]
    return D, I, kth
